# initial kernel scaffold (unmeasured)
import jax
import jax.numpy as jnp
from jax import lax
from jax.experimental import pallas as pl
from jax.experimental.pallas import tpu as pltpu


def kernel(
    x,
):
    def body(*refs):
        pass

    out_shape = jax.ShapeDtypeStruct(..., jnp.float32)
    return pl.pallas_call(body, out_shape=out_shape)(...)


# baseline (device time: 108189 ns/iter reference)
import jax
import jax.numpy as jnp
from jax import lax
from jax.experimental import pallas as pl
from jax.experimental.pallas import tpu as pltpu

HALF = 2048


def kernel(x):
    m_per, n = x.shape

    def body(x_ref, out_ref, send_sem_y, recv_sem_y, send_sem_x, recv_sem_x):
        my_x = lax.axis_index("x")
        my_y = lax.axis_index("y")
        y_nbr = (my_x, 1 - my_y)
        x_nbr = (1 - my_x, my_y)

        barrier = pltpu.get_barrier_semaphore()
        for nbr in (y_nbr, x_nbr):
            pl.semaphore_signal(
                barrier, inc=1, device_id=nbr,
                device_id_type=pl.DeviceIdType.MESH,
            )
        pl.semaphore_wait(barrier, 2)

        out_ref[pl.ds(my_y * m_per, m_per), :] = x_ref[:, :].astype(jnp.bfloat16)

        send_row = my_y * m_per + my_x * HALF
        recv_y_row = (1 - my_y) * m_per + my_x * HALF
        recv_x_row = (1 - my_y) * m_per + (1 - my_x) * HALF

        send_y = pltpu.make_async_remote_copy(
            src_ref=out_ref.at[pl.ds(send_row, HALF), :],
            dst_ref=out_ref.at[pl.ds(send_row, HALF), :],
            send_sem=send_sem_y,
            recv_sem=recv_sem_y,
            device_id=y_nbr,
            device_id_type=pl.DeviceIdType.MESH,
        )
        send_y.start()

        recv_y = pltpu.make_async_remote_copy(
            src_ref=out_ref.at[pl.ds(recv_y_row, HALF), :],
            dst_ref=out_ref.at[pl.ds(recv_y_row, HALF), :],
            send_sem=send_sem_y,
            recv_sem=recv_sem_y,
            device_id=y_nbr,
            device_id_type=pl.DeviceIdType.MESH,
        )
        recv_y.wait_recv()

        send_x = pltpu.make_async_remote_copy(
            src_ref=out_ref.at[pl.ds(recv_y_row, HALF), :],
            dst_ref=out_ref.at[pl.ds(recv_y_row, HALF), :],
            send_sem=send_sem_x,
            recv_sem=recv_sem_x,
            device_id=x_nbr,
            device_id_type=pl.DeviceIdType.MESH,
        )
        send_x.start()

        recv_x = pltpu.make_async_remote_copy(
            src_ref=out_ref.at[pl.ds(recv_x_row, HALF), :],
            dst_ref=out_ref.at[pl.ds(recv_x_row, HALF), :],
            send_sem=send_sem_x,
            recv_sem=recv_sem_x,
            device_id=x_nbr,
            device_id_type=pl.DeviceIdType.MESH,
        )
        recv_x.wait_recv()

        send_y.wait_send()
        send_x.wait_send()

    return pl.pallas_call(
        body,
        out_shape=jax.ShapeDtypeStruct((2 * m_per, n), jnp.bfloat16),
        in_specs=[pl.BlockSpec(memory_space=pltpu.VMEM)],
        out_specs=pl.BlockSpec(memory_space=pltpu.VMEM),
        scratch_shapes=[
            pltpu.SemaphoreType.DMA,
            pltpu.SemaphoreType.DMA,
            pltpu.SemaphoreType.DMA,
            pltpu.SemaphoreType.DMA,
        ],
        compiler_params=pltpu.CompilerParams(collective_id=0),
    )(x)


# device time: 65881 ns/iter; 1.6422x vs baseline; 1.6422x over previous
import jax
import jax.numpy as jnp
from jax import lax
from jax.experimental import pallas as pl
from jax.experimental.pallas import tpu as pltpu

HALF = 2048
N_CHUNK = 16
CH = HALF // N_CHUNK


def kernel(x):
    m_per, n = x.shape

    def body(x_ref, out_ref, send_sems_y, recv_sems_y, send_sems_x, recv_sems_x):
        my_x = lax.axis_index("x")
        my_y = lax.axis_index("y")
        y_nbr = (my_x, 1 - my_y)
        x_nbr = (1 - my_x, my_y)

        barrier = pltpu.get_barrier_semaphore()
        for nbr in (y_nbr, x_nbr):
            pl.semaphore_signal(
                barrier, inc=1, device_id=nbr,
                device_id_type=pl.DeviceIdType.MESH,
            )
        pl.semaphore_wait(barrier, 2)

        send_row = my_y * m_per + my_x * HALF
        recv_y_row = (1 - my_y) * m_per + my_x * HALF
        recv_x_row = (1 - my_y) * m_per + (1 - my_x) * HALF

        def rdma(row, ch, send_sem, recv_sem, dev):
            return pltpu.make_async_remote_copy(
                src_ref=out_ref.at[pl.ds(row, ch), :],
                dst_ref=out_ref.at[pl.ds(row, ch), :],
                send_sem=send_sem,
                recv_sem=recv_sem,
                device_id=dev,
                device_id_type=pl.DeviceIdType.MESH,
            )

        sends_y = []
        for k in range(N_CHUNK):
            src_row = my_x * HALF + k * CH
            out_ref[pl.ds(send_row + k * CH, CH), :] = (
                x_ref[pl.ds(src_row, CH), :].astype(jnp.bfloat16)
            )
            s = rdma(send_row + k * CH, CH, send_sems_y.at[k],
                     recv_sems_y.at[k], y_nbr)
            s.start()
            sends_y.append(s)

        other_row = (1 - my_x) * HALF
        out_ref[pl.ds(my_y * m_per + other_row, HALF), :] = (
            x_ref[pl.ds(other_row, HALF), :].astype(jnp.bfloat16)
        )

        sends_x = []
        for k in range(N_CHUNK):
            row = recv_y_row + k * CH
            rdma(row, CH, send_sems_y.at[k], recv_sems_y.at[k],
                 y_nbr).wait_recv()
            s = rdma(row, CH, send_sems_x.at[k], recv_sems_x.at[k], x_nbr)
            s.start()
            sends_x.append(s)

        for k in range(N_CHUNK):
            rdma(recv_x_row + k * CH, CH, send_sems_x.at[k],
                 recv_sems_x.at[k], x_nbr).wait_recv()
        for s in sends_y:
            s.wait_send()
        for s in sends_x:
            s.wait_send()

    return pl.pallas_call(
        body,
        out_shape=jax.ShapeDtypeStruct((2 * m_per, n), jnp.bfloat16),
        in_specs=[pl.BlockSpec(memory_space=pltpu.VMEM)],
        out_specs=pl.BlockSpec(memory_space=pltpu.VMEM),
        scratch_shapes=[
            pltpu.SemaphoreType.DMA((N_CHUNK,)),
            pltpu.SemaphoreType.DMA((N_CHUNK,)),
            pltpu.SemaphoreType.DMA((N_CHUNK,)),
            pltpu.SemaphoreType.DMA((N_CHUNK,)),
        ],
        compiler_params=pltpu.CompilerParams(collective_id=0),
    )(x)


# device time: 65852 ns/iter; 1.6429x vs baseline; 1.0004x over previous
import jax
import jax.numpy as jnp
from jax import lax
from jax.experimental import pallas as pl
from jax.experimental.pallas import tpu as pltpu

HALF = 2048
CHUNK_ROWS = [128] * 15 + [64, 64]
CHUNK_OFFS = [sum(CHUNK_ROWS[:k]) for k in range(len(CHUNK_ROWS))]
N_CHUNK = len(CHUNK_ROWS)
assert sum(CHUNK_ROWS) == HALF


def kernel(x):
    m_per, n = x.shape

    def body(x_ref, out_ref, mine_ref, send_sems_y, recv_sems_y,
             send_sems_x, recv_sems_x, local_sem):
        my_x = lax.axis_index("x")
        my_y = lax.axis_index("y")
        y_nbr = (my_x, 1 - my_y)
        x_nbr = (1 - my_x, my_y)

        barrier = pltpu.get_barrier_semaphore()
        for nbr in (y_nbr, x_nbr):
            pl.semaphore_signal(
                barrier, inc=1, device_id=nbr,
                device_id_type=pl.DeviceIdType.MESH,
            )
        pl.semaphore_wait(barrier, 2)

        send_row = my_y * m_per + my_x * HALF
        recv_y_row = (1 - my_y) * m_per + my_x * HALF
        recv_x_row = (1 - my_y) * m_per + (1 - my_x) * HALF

        def rdma(src, row, ch, send_sem, recv_sem, dev):
            return pltpu.make_async_remote_copy(
                src_ref=src,
                dst_ref=out_ref.at[pl.ds(row, ch), :],
                send_sem=send_sem,
                recv_sem=recv_sem,
                device_id=dev,
                device_id_type=pl.DeviceIdType.MESH,
            )

        sends_y = []
        for k in range(N_CHUNK):
            off, ch = CHUNK_OFFS[k], CHUNK_ROWS[k]
            src_row = my_x * HALF + off
            mine_ref[pl.ds(src_row, ch), :] = (
                x_ref[pl.ds(src_row, ch), :].astype(jnp.bfloat16)
            )
            s = rdma(mine_ref.at[pl.ds(src_row, ch), :], send_row + off,
                     ch, send_sems_y.at[k], recv_sems_y.at[k], y_nbr)
            s.start()
            sends_y.append(s)

        other_row = (1 - my_x) * HALF
        mine_ref[pl.ds(other_row, HALF), :] = (
            x_ref[pl.ds(other_row, HALF), :].astype(jnp.bfloat16)
        )
        local_copy = pltpu.make_async_copy(
            mine_ref,
            out_ref.at[pl.ds(my_y * m_per, m_per), :],
            local_sem,
        )
        local_copy.start()

        sends_x = []
        for k in range(N_CHUNK):
            off, ch = CHUNK_OFFS[k], CHUNK_ROWS[k]
            row = recv_y_row + off
            rdma(out_ref.at[pl.ds(row, ch), :], row, ch,
                 send_sems_y.at[k], recv_sems_y.at[k], y_nbr).wait_recv()
            s = rdma(out_ref.at[pl.ds(row, ch), :], row, ch,
                     send_sems_x.at[k], recv_sems_x.at[k], x_nbr)
            s.start()
            sends_x.append(s)

        for k in range(N_CHUNK):
            off, ch = CHUNK_OFFS[k], CHUNK_ROWS[k]
            row = recv_x_row + off
            rdma(out_ref.at[pl.ds(row, ch), :], row, ch,
                 send_sems_x.at[k], recv_sems_x.at[k], x_nbr).wait_recv()
        local_copy.wait()
        for s in sends_y:
            s.wait_send()
        for s in sends_x:
            s.wait_send()

    return pl.pallas_call(
        body,
        out_shape=jax.ShapeDtypeStruct((2 * m_per, n), jnp.bfloat16),
        in_specs=[pl.BlockSpec(memory_space=pltpu.VMEM)],
        out_specs=pl.BlockSpec(memory_space=pl.ANY),
        scratch_shapes=[
            pltpu.VMEM((m_per, n), jnp.bfloat16),
            pltpu.SemaphoreType.DMA((N_CHUNK,)),
            pltpu.SemaphoreType.DMA((N_CHUNK,)),
            pltpu.SemaphoreType.DMA((N_CHUNK,)),
            pltpu.SemaphoreType.DMA((N_CHUNK,)),
            pltpu.SemaphoreType.DMA,
        ],
        compiler_params=pltpu.CompilerParams(collective_id=0),
    )(x)


# device time: 65839 ns/iter; 1.6432x vs baseline; 1.0002x over previous
import jax
import jax.numpy as jnp
from jax import lax
from jax.experimental import pallas as pl
from jax.experimental.pallas import tpu as pltpu

HALF = 2048
N_CHUNK = 16
CH = HALF // N_CHUNK


def kernel(x):
    m_per, n = x.shape

    def body(x_ref, out_ref, mine_ref, send_sems_y, recv_sems_y,
             send_sems_x, recv_sems_x, local_sem):
        my_x = lax.axis_index("x")
        my_y = lax.axis_index("y")
        y_nbr = (my_x, 1 - my_y)
        x_nbr = (1 - my_x, my_y)

        barrier = pltpu.get_barrier_semaphore()
        for nbr in (y_nbr, x_nbr):
            pl.semaphore_signal(
                barrier, inc=1, device_id=nbr,
                device_id_type=pl.DeviceIdType.MESH,
            )
        pl.semaphore_wait(barrier, 2)

        send_row = my_y * m_per + my_x * HALF
        recv_y_row = (1 - my_y) * m_per + my_x * HALF
        recv_x_row = (1 - my_y) * m_per + (1 - my_x) * HALF

        def rdma(src, row, ch, send_sem, recv_sem, dev):
            return pltpu.make_async_remote_copy(
                src_ref=src,
                dst_ref=out_ref.at[pl.ds(row, ch), :],
                send_sem=send_sem,
                recv_sem=recv_sem,
                device_id=dev,
                device_id_type=pl.DeviceIdType.MESH,
            )

        sends_y = []
        for k in range(N_CHUNK):
            src_row = my_x * HALF + k * CH
            mine_ref[pl.ds(src_row, CH), :] = (
                x_ref[pl.ds(src_row, CH), :].astype(jnp.bfloat16)
            )
            s = rdma(mine_ref.at[pl.ds(src_row, CH), :], send_row + k * CH,
                     CH, send_sems_y.at[k], recv_sems_y.at[k], y_nbr)
            s.start()
            sends_y.append(s)

        other_row = (1 - my_x) * HALF
        mine_ref[pl.ds(other_row, HALF), :] = (
            x_ref[pl.ds(other_row, HALF), :].astype(jnp.bfloat16)
        )
        local_copy = pltpu.make_async_copy(
            mine_ref,
            out_ref.at[pl.ds(my_y * m_per, m_per), :],
            local_sem,
        )
        local_copy.start()

        sends_x = []
        for k in range(N_CHUNK):
            row = recv_y_row + k * CH
            rdma(out_ref.at[pl.ds(row, CH), :], row, CH,
                 send_sems_y.at[k], recv_sems_y.at[k], y_nbr).wait_recv()
            s = rdma(out_ref.at[pl.ds(row, CH), :], row, CH,
                     send_sems_x.at[k], recv_sems_x.at[k], x_nbr)
            s.start()
            sends_x.append(s)

        for k in range(N_CHUNK):
            row = recv_x_row + k * CH
            rdma(out_ref.at[pl.ds(row, CH), :], row, CH,
                 send_sems_x.at[k], recv_sems_x.at[k], x_nbr).wait_recv()
        local_copy.wait()
        for s in sends_y:
            s.wait_send()
        for s in sends_x:
            s.wait_send()

    return pl.pallas_call(
        body,
        out_shape=jax.ShapeDtypeStruct((2 * m_per, n), jnp.bfloat16),
        in_specs=[pl.BlockSpec(memory_space=pltpu.VMEM)],
        out_specs=pl.BlockSpec(memory_space=pl.ANY),
        scratch_shapes=[
            pltpu.VMEM((m_per, n), jnp.bfloat16),
            pltpu.SemaphoreType.DMA((N_CHUNK,)),
            pltpu.SemaphoreType.DMA((N_CHUNK,)),
            pltpu.SemaphoreType.DMA((N_CHUNK,)),
            pltpu.SemaphoreType.DMA((N_CHUNK,)),
            pltpu.SemaphoreType.DMA,
        ],
        compiler_params=pltpu.CompilerParams(collective_id=0),
    )(x)
